# Initial kernel scaffold; baseline (speedup 1.0000x reference)
import jax
import jax.numpy as jnp
from jax.experimental import pallas as pl


def kernel(x, W_base, b_base, Wd, bd, Wu, bu, We, be, Wo, bo, conn_idx):
    raise NotImplementedError("write your pallas kernel here")


# speedup vs baseline: 1.4238x; 1.4238x over previous
import functools

import jax
import jax.numpy as jnp
from jax import lax
from jax.experimental import pallas as pl
from jax.experimental.pallas import tpu as pltpu
from jax.experimental.pallas import tpu_sc as plsc

_INTERPRET = False

F32 = jnp.float32


def _disc_body(E, H, x_ref, wcat_ref, be_ref, out_ref, hth_ref, hp_ref, g_ref):
    EH = E * H
    b = pl.program_id(0)
    t = pl.program_id(1)
    nt = pl.num_programs(1)

    xb = x_ref[0]
    res = jnp.dot(xb, wcat_ref[...], preferred_element_type=F32)
    hb = jnp.maximum(res[:, :EH] + be_ref[...], 0.0)
    p = res[:, EH:]

    @pl.when((b == 0) & (t == 0))
    def _():
        wot = wcat_ref[:, EH:]
        g_ref[...] = lax.dot_general(
            wot, wot, (((0,), (0,)), ((), ())), preferred_element_type=F32
        )

    @pl.when(t == 0)
    def _():
        hth_ref[...] = jnp.zeros_like(hth_ref)
        hp_ref[...] = jnp.zeros_like(hp_ref)

    hth_ref[...] += lax.dot_general(
        hb, hb, (((0,), (0,)), ((), ())), preferred_element_type=F32
    )
    hp_ref[...] += jnp.sum(hb * p, axis=0, keepdims=True)

    @pl.when(t == nt - 1)
    def _():
        r = lax.broadcasted_iota(jnp.int32, (EH, EH), 0) // H
        c = lax.broadcasted_iota(jnp.int32, (EH, EH), 1) // H
        masked = jnp.where(r == c, hth_ref[...] * g_ref[...], 0.0)
        colsum = jnp.sum(masked, axis=0, keepdims=True)
        ind = (
            lax.broadcasted_iota(jnp.int32, (EH, 128), 0) // H
            == lax.broadcasted_iota(jnp.int32, (EH, 128), 1)
        ).astype(F32)
        t1 = jnp.dot(colsum, ind, preferred_element_type=F32)
        hpg = jnp.dot(hp_ref[...], ind, preferred_element_type=F32)
        lane = lax.broadcasted_iota(jnp.int32, (1, 128), 1)
        out_ref[0] = jnp.where(lane < E, t1 - 2.0 * hpg, jnp.float32(jnp.inf))


def _disc_call(x, We, be, Wo, TB):
    B, T, D = x.shape
    E, _, H = We.shape
    EH = E * H
    NT = T // TB
    we_flat = jnp.transpose(We, (1, 0, 2)).reshape(D, EH)
    wot_flat = jnp.transpose(Wo, (2, 0, 1)).reshape(D, EH)
    wcat = jnp.concatenate([we_flat, wot_flat], axis=1)
    be_flat = be.reshape(1, EH)

    return pl.pallas_call(
        functools.partial(_disc_body, E, H),
        grid=(B, NT),
        in_specs=[
            pl.BlockSpec((1, TB, D), lambda b, t: (b, t, 0)),
            pl.BlockSpec((D, 2 * EH), lambda b, t: (0, 0)),
            pl.BlockSpec((1, EH), lambda b, t: (0, 0)),
        ],
        out_specs=pl.BlockSpec((1, 1, 128), lambda b, t: (b, 0, 0)),
        out_shape=jax.ShapeDtypeStruct((B, 1, 128), F32),
        scratch_shapes=[
            pltpu.VMEM((EH, EH), F32),
            pltpu.VMEM((1, EH), F32),
            pltpu.VMEM((EH, EH), F32),
        ],
        compiler_params=pltpu.CompilerParams(
            dimension_semantics=("arbitrary", "arbitrary"),
        ),
        interpret=_INTERPRET,
    )(x, wcat, be_flat)


def _routing_body(B, E, losses_hbm, conn_hbm, out_hbm, loss_v, conn_v, out_v):
    c = lax.axis_index("c")
    s = lax.axis_index("s")

    @pl.when((c == 0) & (s == 0))
    def _():
        pltpu.sync_copy(losses_hbm, loss_v)
        pltpu.sync_copy(conn_hbm, conn_v)
        conn_row = conn_v[...]
        for b in range(B):
            row = loss_v[b, 0:16]
            best = row[0]
            best_conn = conn_row[0]
            for e in range(1, E):
                le = row[e]
                pred = le < best
                best = jnp.where(pred, le, best)
                best_conn = jnp.where(pred, conn_row[e], best_conn)
            out_v[b] = lax.broadcast(best_conn, (16,))
        pltpu.sync_copy(out_v, out_hbm)


def _routing_call(losses, conn16):
    B = losses.shape[0]
    mesh = plsc.VectorSubcoreMesh(core_axis_name="c", subcore_axis_name="s")
    return pl.kernel(
        functools.partial(_routing_body, B, 8),
        out_type=jax.ShapeDtypeStruct((B, 16), jnp.int32),
        mesh=mesh,
        scratch_types=[
            pltpu.VMEM((B, 128), F32),
            pltpu.VMEM((16,), jnp.int32),
            pltpu.VMEM((B, 16), jnp.int32),
        ],
    )(losses, conn16)


def _adapter_body(aidx_ref, x_ref, wb_ref, bb_ref, wd_ref, bd_ref, wu_ref,
                  bu_ref, out_ref):
    xb = x_ref[0]
    base = jnp.dot(xb, wb_ref[...], preferred_element_type=F32)
    hid = jnp.maximum(
        jnp.dot(xb, wd_ref[0], preferred_element_type=F32) + bd_ref[0], 0.0
    )
    ad = jnp.dot(hid, wu_ref[0], preferred_element_type=F32)
    out_ref[0] = base + ad + bb_ref[...] + bu_ref[0]


def _adapter_call(aidx, x, W_base, b_base, Wd, bd, Wu, bu, TB):
    B, T, D = x.shape
    E, _, R = Wd.shape
    NT = T // TB

    grid_spec = pltpu.PrefetchScalarGridSpec(
        num_scalar_prefetch=1,
        grid=(B, NT),
        in_specs=[
            pl.BlockSpec((1, TB, D), lambda b, t, a: (b, t, 0)),
            pl.BlockSpec((D, D), lambda b, t, a: (0, 0)),
            pl.BlockSpec((1, D), lambda b, t, a: (0, 0)),
            pl.BlockSpec((1, D, R), lambda b, t, a: (a[b], 0, 0)),
            pl.BlockSpec((1, 1, R), lambda b, t, a: (a[b], 0, 0)),
            pl.BlockSpec((1, R, D), lambda b, t, a: (a[b], 0, 0)),
            pl.BlockSpec((1, 1, D), lambda b, t, a: (a[b], 0, 0)),
        ],
        out_specs=pl.BlockSpec((1, TB, D), lambda b, t, a: (b, t, 0)),
    )
    return pl.pallas_call(
        _adapter_body,
        grid_spec=grid_spec,
        out_shape=jax.ShapeDtypeStruct((B, T, D), F32),
        compiler_params=pltpu.CompilerParams(
            dimension_semantics=("arbitrary", "arbitrary"),
        ),
        interpret=_INTERPRET,
    )(aidx, x, W_base, b_base.reshape(1, D), Wd, bd.reshape(E, 1, R), Wu,
      bu.reshape(E, 1, D))


def kernel(x, W_base, b_base, Wd, bd, Wu, bu, We, be, Wo, bo, conn_idx):
    B, T, D = x.shape
    E = We.shape[0]
    TB = 512

    losses = _disc_call(x, We, be, Wo, TB).reshape(B, 128)
    conn16 = jnp.concatenate(
        [conn_idx.astype(jnp.int32), jnp.zeros((16 - E,), jnp.int32)]
    )
    aidx = _routing_call(losses, conn16)[:, 0]
    return _adapter_call(aidx, x, W_base, b_base, Wd, bd, Wu, bu, TB)
